# baseline (speedup 1.0000x reference)
import functools

import jax
import jax.numpy as jnp
from jax import lax
from jax.experimental import pallas as pl
from jax.experimental.pallas import tpu as pltpu
from jax.experimental.pallas import tpu_sc as plsc

NC = 2
NS = 16
NW = NC * NS
LB = 20
B = 1 << LB
NP = B
W = NP // NW
CH = 8192
NCHUNK = W // CH
ROWS = CH // 128
VECS = CH // 16

CNT_SHIFT = 25
VAL_SCALE = float(1 << 16)
INV_VAL_SCALE = 1.0 / (1 << 16)

RB = 128
NB = B // (RB * 128)


def _sc_hist(n_real, yt_hbm, yp_hbm, padv_hbm, padz_hbm, t_hbm,
             yt_a, yt_b, yp_a, yp_b, idx3, val3, t_sh, sem_in, scat_sem):
    yt_bufs = (yt_a, yt_b)
    yp_bufs = (yp_a, yp_b)
    c = lax.axis_index("c")
    s = lax.axis_index("s")
    wid = s * NC + c
    base = wid * W
    n0 = (n_real // CH) * CH
    nmix = n_real - n0

    def _zv(i, _):
        val3[pl.ds(i * 16, 16)] = jnp.zeros((16,), jnp.int32)
        return _
    lax.fori_loop(0, VECS, _zv, None)
    stripe = s * (B // NS)
    for j in range(B // NS // CH):
        pltpu.sync_copy(val3, t_sh.at[pl.ds(stripe + j * CH, CH)])
    plsc.subcore_barrier()

    def _fire_in(k):
        b = k % 2
        off = base + k * CH

        @pl.when(off + CH <= n_real)
        def _():
            pltpu.async_copy(yt_hbm.at[pl.ds(off, CH)], yt_bufs[b], sem_in)
            pltpu.async_copy(yp_hbm.at[pl.ds(off, CH)], yp_bufs[b], sem_in)

        @pl.when(off >= n_real)
        def _():
            po = off - n_real
            pltpu.async_copy(padv_hbm.at[pl.ds(po, CH)], yt_bufs[b], sem_in)
            pltpu.async_copy(padz_hbm.at[pl.ds(po, CH)], yp_bufs[b], sem_in)

        @pl.when((off < n_real) & (off + CH > n_real))
        def _():
            pltpu.async_copy(yt_hbm.at[pl.ds(n0, nmix)],
                             yt_bufs[b].at[pl.ds(0, nmix)], sem_in)
            pltpu.async_copy(padv_hbm.at[pl.ds(0, CH - nmix)],
                             yt_bufs[b].at[pl.ds(nmix, CH - nmix)], sem_in)
            pltpu.async_copy(yp_hbm.at[pl.ds(n0, nmix)],
                             yp_bufs[b].at[pl.ds(0, nmix)], sem_in)
            pltpu.async_copy(padz_hbm.at[pl.ds(0, CH - nmix)],
                             yp_bufs[b].at[pl.ds(nmix, CH - nmix)], sem_in)

    _fire_in(0)
    for k in range(NCHUNK):
        b = k % 2
        pltpu.make_async_copy(yt_hbm.at[pl.ds(0, CH)], yt_bufs[b],
                              sem_in).wait()
        pltpu.make_async_copy(yt_hbm.at[pl.ds(0, CH)], yp_bufs[b],
                              sem_in).wait()
        if k + 1 < NCHUNK:
            _fire_in(k + 1)

        def _row(r, _):
            for u in range(8):
                sl = pl.ds(r * 128 + u * 16, 16)
                t = yt_bufs[b][sl]
                q = jnp.minimum((t * float(B)).astype(jnp.int32), B - 1)
                idx3[r, pl.ds(u * 16, 16)] = q
                p = yp_bufs[b][sl]
                ps = p * VAL_SCALE
                half = jnp.where(ps >= 0.0, 0.5, -0.5)
                v = (ps + half).astype(jnp.int32) + (1 << CNT_SHIFT)
                val3[sl] = v
            pltpu.async_copy(val3.at[pl.ds(r * 128, 128)],
                             t_sh.at[idx3.at[r]], scat_sem, add=True)
            return _
        lax.fori_loop(0, ROWS, _row, None)

        pltpu.make_async_copy(yt_hbm.at[pl.ds(0, CH)], yt_bufs[b],
                              scat_sem).wait()

    plsc.subcore_barrier()
    off_out = c * B + stripe
    pltpu.sync_copy(t_sh.at[pl.ds(stripe, B // NS)],
                    t_hbm.at[pl.ds(off_out, B // NS)])


def _cumsum(x, axis):
    n = x.shape[axis]
    k = 1
    while k < n:
        shp = list(x.shape)
        shp[axis] = k
        shifted = jnp.concatenate(
            [jnp.zeros(shp, x.dtype), lax.slice_in_dim(x, 0, n - k, axis=axis)],
            axis=axis)
        x = x + shifted
        k *= 2
    return x


def _tc_reduce(n_real, t_ref, out_ref, st_ref):
    g = pl.program_id(0)

    @pl.when(g == 0)
    def _():
        st_ref[0] = 0.0
        st_ref[1] = 0.0

    t0 = t_ref[0, 0]
    t1 = t_ref[1, 0]
    c0 = (t0 + (1 << (CNT_SHIFT - 1))) >> CNT_SHIFT
    c1 = (t1 + (1 << (CNT_SHIFT - 1))) >> CNT_SHIFT
    f0 = t0 - (c0 << CNT_SHIFT)
    f1 = t1 - (c1 << CNT_SHIFT)
    cnt = (c0 + c1).astype(jnp.float32)
    S = (f0 + f1).astype(jnp.float32) * INV_VAL_SCALE
    gi = (g * RB * 128
          + lax.broadcasted_iota(jnp.int32, (RB, 128), 0) * 128
          + lax.broadcasted_iota(jnp.int32, (RB, 128), 1))
    cnt = cnt - jnp.where(gi >= n_real, 1.0, 0.0)

    ia = lax.broadcasted_iota(jnp.int32, (RB, 128), 0)
    ib = lax.broadcasted_iota(jnp.int32, (RB, 128), 1)
    ut = (ia < ib).astype(jnp.float32)
    lt = (ia > ib).astype(jnp.float32)
    colpre = lax.dot_general(cnt, ut, (((1,), (0,)), ((), ())),
                             preferred_element_type=jnp.float32)
    rowsum = jnp.sum(cnt, axis=1, keepdims=True)
    rowpre = lax.dot_general(lt, rowsum, (((1,), (0,)), ((), ())),
                             preferred_element_type=jnp.float32)
    carry = st_ref[0]
    terms = (2.0 * (carry + rowpre + colpre) + cnt - float(n_real)) * S
    st_ref[0] = carry + jnp.sum(rowsum)
    st_ref[1] = st_ref[1] + jnp.sum(terms)

    @pl.when(g == NB - 1)
    def _():
        out_ref[0, 0] = -st_ref[1] * float(1.0 / (n_real * n_real))


def kernel(y_pred, y_true):
    n = y_pred.shape[0]
    y_true = y_true.reshape(y_pred.shape)
    pad = NP - n
    pad_g = jnp.arange(n, NP, dtype=jnp.int32)
    pad_vals = (pad_g.astype(jnp.float32) + 0.5) * (1.0 / B)
    pad_zeros = jnp.zeros((pad,), jnp.float32)

    mesh = plsc.VectorSubcoreMesh(core_axis_name="c", subcore_axis_name="s",
                                  num_cores=NC, num_subcores=NS)
    hist = pl.kernel(
        functools.partial(_sc_hist, n),
        out_type=jax.ShapeDtypeStruct((NC * B,), jnp.int32),
        mesh=mesh,
        scratch_types=[
            pltpu.VMEM((CH,), jnp.float32),
            pltpu.VMEM((CH,), jnp.float32),
            pltpu.VMEM((CH,), jnp.float32),
            pltpu.VMEM((CH,), jnp.float32),
            pltpu.VMEM((ROWS, 128), jnp.int32),
            pltpu.VMEM((CH,), jnp.int32),
            pltpu.VMEM_SHARED((B,), jnp.int32),
            pltpu.SemaphoreType.DMA,
            pltpu.SemaphoreType.DMA,
        ],
    )
    tpk = hist(y_true, y_pred, pad_vals, pad_zeros)

    t4 = tpk.reshape(NC, NB, RB, 128)
    out = pl.pallas_call(
        functools.partial(_tc_reduce, n),
        grid=(NB,),
        in_specs=[pl.BlockSpec((NC, 1, RB, 128), lambda g: (0, g, 0, 0))],
        out_specs=pl.BlockSpec((1, 1), lambda g: (0, 0),
                               memory_space=pltpu.SMEM),
        out_shape=jax.ShapeDtypeStruct((1, 1), jnp.float32),
        scratch_shapes=[pltpu.SMEM((2,), jnp.float32)],
        compiler_params=pltpu.CompilerParams(
            dimension_semantics=("arbitrary",)),
    )(t4)
    return out[0, 0]


# speedup vs baseline: 1.9337x; 1.9337x over previous
import functools

import jax
import jax.numpy as jnp
from jax import lax
from jax.experimental import pallas as pl
from jax.experimental.pallas import tpu as pltpu
from jax.experimental.pallas import tpu_sc as plsc

NC = 2
NS = 16
NW = NC * NS
LB = 19
B = 1 << LB
NP = 2 * B
W = NP // NW
CH = 8192
NCHUNK = W // CH
ROWS = CH // 128
VECS = CH // 16

CNT_SHIFT = 25
VAL_SCALE = float(1 << 16)
INV_VAL_SCALE = 1.0 / (1 << 16)

RB = 1024
NB = B // (RB * 128)


def _sc_hist(n_real, yt_hbm, yp_hbm, padv_hbm, padz_hbm, t_hbm,
             yt_a, yt_b, yp_a, yp_b, idx3, val3, t_sh, sem_in, scat_sem):
    yt_bufs = (yt_a, yt_b)
    yp_bufs = (yp_a, yp_b)
    c = lax.axis_index("c")
    s = lax.axis_index("s")
    wid = s * NC + c
    base = wid * W
    n0 = (n_real // CH) * CH
    nmix = n_real - n0

    def _fire_in(k):
        b = k % 2
        off = base + k * CH

        @pl.when(off + CH <= n_real)
        def _():
            pltpu.async_copy(yt_hbm.at[pl.ds(off, CH)], yt_bufs[b], sem_in)
            pltpu.async_copy(yp_hbm.at[pl.ds(off, CH)], yp_bufs[b], sem_in)

        @pl.when(off >= n_real)
        def _():
            po = off - n_real
            pltpu.async_copy(padv_hbm.at[pl.ds(po, CH)], yt_bufs[b], sem_in)
            pltpu.async_copy(padz_hbm.at[pl.ds(po, CH)], yp_bufs[b], sem_in)

        @pl.when((off < n_real) & (off + CH > n_real))
        def _():
            pltpu.async_copy(yt_hbm.at[pl.ds(n0, nmix)],
                             yt_bufs[b].at[pl.ds(0, nmix)], sem_in)
            pltpu.async_copy(padv_hbm.at[pl.ds(0, CH - nmix)],
                             yt_bufs[b].at[pl.ds(nmix, CH - nmix)], sem_in)
            pltpu.async_copy(yp_hbm.at[pl.ds(n0, nmix)],
                             yp_bufs[b].at[pl.ds(0, nmix)], sem_in)
            pltpu.async_copy(padz_hbm.at[pl.ds(0, CH - nmix)],
                             yp_bufs[b].at[pl.ds(nmix, CH - nmix)], sem_in)

    _fire_in(0)

    @plsc.parallel_loop(0, VECS, unroll=4)
    def _zv(i):
        val3[pl.ds(i * 16, 16)] = jnp.zeros((16,), jnp.int32)
    stripe = s * (B // NS)
    for j in range(B // NS // CH):
        pltpu.sync_copy(val3, t_sh.at[pl.ds(stripe + j * CH, CH)])
    plsc.subcore_barrier()

    for k in range(NCHUNK):
        b = k % 2
        pltpu.make_async_copy(yt_hbm.at[pl.ds(0, CH)], yt_bufs[b],
                              sem_in).wait()
        pltpu.make_async_copy(yt_hbm.at[pl.ds(0, CH)], yp_bufs[b],
                              sem_in).wait()
        if k + 1 < NCHUNK:
            _fire_in(k + 1)

        @plsc.parallel_loop(0, ROWS, unroll=2)
        def _row(r):
            for u in range(8):
                sl = pl.ds(r * 128 + u * 16, 16)
                t = yt_bufs[b][sl]
                q = jnp.minimum((t * float(B)).astype(jnp.int32), B - 1)
                idx3[r, pl.ds(u * 16, 16)] = q
                p = yp_bufs[b][sl]
                ps = p * VAL_SCALE
                half = jnp.where(ps >= 0.0, 0.5, -0.5)
                v = (ps + half).astype(jnp.int32) + (1 << CNT_SHIFT)
                val3[sl] = v
            pltpu.async_copy(val3.at[pl.ds(r * 128, 128)],
                             t_sh.at[idx3.at[r]], scat_sem, add=True)

        pltpu.make_async_copy(yt_hbm.at[pl.ds(0, CH)], yt_bufs[b],
                              scat_sem).wait()

    plsc.subcore_barrier()
    off_out = c * B + stripe
    pltpu.sync_copy(t_sh.at[pl.ds(stripe, B // NS)],
                    t_hbm.at[pl.ds(off_out, B // NS)])


def _cumsum(x, axis):
    n = x.shape[axis]
    k = 1
    while k < n:
        shp = list(x.shape)
        shp[axis] = k
        shifted = jnp.concatenate(
            [jnp.zeros(shp, x.dtype), lax.slice_in_dim(x, 0, n - k, axis=axis)],
            axis=axis)
        x = x + shifted
        k *= 2
    return x


def _tc_reduce(n_real, t_ref, out_ref, st_ref):
    g = pl.program_id(0)

    @pl.when(g == 0)
    def _():
        st_ref[0] = 0.0
        st_ref[1] = 0.0

    t0 = t_ref[0, 0]
    t1 = t_ref[1, 0]
    c0 = (t0 + (1 << (CNT_SHIFT - 1))) >> CNT_SHIFT
    c1 = (t1 + (1 << (CNT_SHIFT - 1))) >> CNT_SHIFT
    f0 = t0 - (c0 << CNT_SHIFT)
    f1 = t1 - (c1 << CNT_SHIFT)
    cnt = (c0 + c1).astype(jnp.float32)
    S = (f0 + f1).astype(jnp.float32) * INV_VAL_SCALE
    gi = (g * RB * 128
          + lax.broadcasted_iota(jnp.int32, (RB, 128), 0) * 128
          + lax.broadcasted_iota(jnp.int32, (RB, 128), 1))
    cnt = cnt - jnp.where(gi >= n_real - B, 1.0, 0.0)

    rowsum = jnp.sum(cnt, axis=1, keepdims=True)
    rowpre = _cumsum(rowsum, 0) - rowsum
    colpre = _cumsum(cnt, 1) - cnt
    carry = st_ref[0]
    terms = (2.0 * (carry + rowpre + colpre) + cnt - float(n_real)) * S
    st_ref[0] = carry + jnp.sum(rowsum)
    st_ref[1] = st_ref[1] + jnp.sum(terms)

    @pl.when(g == NB - 1)
    def _():
        out_ref[0, 0] = -st_ref[1] * float(1.0 / (n_real * n_real))


def kernel(y_pred, y_true):
    n = y_pred.shape[0]
    y_true = y_true.reshape(y_pred.shape)
    pad = NP - n
    pad_g = jnp.arange(n, NP, dtype=jnp.int32)
    pad_vals = ((pad_g & (B - 1)).astype(jnp.float32) + 0.5) * (1.0 / B)
    pad_zeros = jnp.zeros((pad,), jnp.float32)

    mesh = plsc.VectorSubcoreMesh(core_axis_name="c", subcore_axis_name="s",
                                  num_cores=NC, num_subcores=NS)
    hist = pl.kernel(
        functools.partial(_sc_hist, n),
        out_type=jax.ShapeDtypeStruct((NC * B,), jnp.int32),
        mesh=mesh,
        scratch_types=[
            pltpu.VMEM((CH,), jnp.float32),
            pltpu.VMEM((CH,), jnp.float32),
            pltpu.VMEM((CH,), jnp.float32),
            pltpu.VMEM((CH,), jnp.float32),
            pltpu.VMEM((ROWS, 128), jnp.int32),
            pltpu.VMEM((CH,), jnp.int32),
            pltpu.VMEM_SHARED((B,), jnp.int32),
            pltpu.SemaphoreType.DMA,
            pltpu.SemaphoreType.DMA,
        ],
    )
    tpk = hist(y_true, y_pred, pad_vals, pad_zeros)

    t4 = tpk.reshape(NC, NB, RB, 128)
    out = pl.pallas_call(
        functools.partial(_tc_reduce, n),
        grid=(NB,),
        in_specs=[pl.BlockSpec((NC, 1, RB, 128), lambda g: (0, g, 0, 0))],
        out_specs=pl.BlockSpec((1, 1), lambda g: (0, 0),
                               memory_space=pltpu.SMEM),
        out_shape=jax.ShapeDtypeStruct((1, 1), jnp.float32),
        scratch_shapes=[pltpu.SMEM((2,), jnp.float32)],
        compiler_params=pltpu.CompilerParams(
            dimension_semantics=("arbitrary",)),
    )(t4)
    return out[0, 0]


# speedup vs baseline: 1.9354x; 1.0009x over previous
import functools

import jax
import jax.numpy as jnp
import numpy as np
from jax import lax
from jax.experimental import pallas as pl
from jax.experimental.pallas import tpu as pltpu
from jax.experimental.pallas import tpu_sc as plsc

NC = 2
NS = 16
NW = NC * NS
LB = 19
B = 1 << LB
NP = 2 * B
W = NP // NW
CH = 8192
NCHUNK = W // CH
ROWS = CH // 128
VECS = CH // 16

CNT_SHIFT = 25
VAL_SCALE = float(1 << 16)
INV_VAL_SCALE = 1.0 / (1 << 16)

RB = 1024
NB = B // (RB * 128)


def _sc_hist(n_real, yt_hbm, yp_hbm, padv_hbm, padz_hbm, t_hbm,
             yt_a, yt_b, yp_a, yp_b, idx3, val3, t_sh, sem_in, scat_sem):
    yt_bufs = (yt_a, yt_b)
    yp_bufs = (yp_a, yp_b)
    c = lax.axis_index("c")
    s = lax.axis_index("s")
    wid = s * NC + c
    base = wid * W
    n0 = (n_real // CH) * CH
    nmix = n_real - n0

    def _fire_in(k):
        b = k % 2
        off = base + k * CH

        @pl.when(off + CH <= n_real)
        def _():
            pltpu.async_copy(yt_hbm.at[pl.ds(off, CH)], yt_bufs[b], sem_in)
            pltpu.async_copy(yp_hbm.at[pl.ds(off, CH)], yp_bufs[b], sem_in)

        @pl.when(off >= n_real)
        def _():
            po = off - n_real
            pltpu.async_copy(padv_hbm.at[pl.ds(po, CH)], yt_bufs[b], sem_in)
            pltpu.async_copy(padz_hbm.at[pl.ds(po, CH)], yp_bufs[b], sem_in)

        @pl.when((off < n_real) & (off + CH > n_real))
        def _():
            pltpu.async_copy(yt_hbm.at[pl.ds(n0, nmix)],
                             yt_bufs[b].at[pl.ds(0, nmix)], sem_in)
            pltpu.async_copy(padv_hbm.at[pl.ds(0, CH - nmix)],
                             yt_bufs[b].at[pl.ds(nmix, CH - nmix)], sem_in)
            pltpu.async_copy(yp_hbm.at[pl.ds(n0, nmix)],
                             yp_bufs[b].at[pl.ds(0, nmix)], sem_in)
            pltpu.async_copy(padz_hbm.at[pl.ds(0, CH - nmix)],
                             yp_bufs[b].at[pl.ds(nmix, CH - nmix)], sem_in)

    _fire_in(0)

    @plsc.parallel_loop(0, VECS, unroll=4)
    def _zv(i):
        val3[pl.ds(i * 16, 16)] = jnp.zeros((16,), jnp.int32)
    stripe = s * (B // NS)
    for j in range(B // NS // CH):
        pltpu.sync_copy(val3, t_sh.at[pl.ds(stripe + j * CH, CH)])
    plsc.subcore_barrier()

    for k in range(NCHUNK):
        b = k % 2
        pltpu.make_async_copy(yt_hbm.at[pl.ds(0, CH)], yt_bufs[b],
                              sem_in).wait()
        pltpu.make_async_copy(yt_hbm.at[pl.ds(0, CH)], yp_bufs[b],
                              sem_in).wait()
        if k + 1 < NCHUNK:
            _fire_in(k + 1)

        @plsc.parallel_loop(0, ROWS, unroll=2)
        def _row(r):
            for u in range(8):
                sl = pl.ds(r * 128 + u * 16, 16)
                t = yt_bufs[b][sl]
                q = jnp.minimum((t * float(B)).astype(jnp.int32), B - 1)
                idx3[r, pl.ds(u * 16, 16)] = q
                p = yp_bufs[b][sl]
                ps = p * VAL_SCALE
                half = jnp.where(ps >= 0.0, 0.5, -0.5)
                v = (ps + half).astype(jnp.int32) + (1 << CNT_SHIFT)
                val3[sl] = v
            pltpu.async_copy(val3.at[pl.ds(r * 128, 128)],
                             t_sh.at[idx3.at[r]], scat_sem, add=True)

        pltpu.make_async_copy(yt_hbm.at[pl.ds(0, CH)], yt_bufs[b],
                              scat_sem).wait()

    plsc.subcore_barrier()
    off_out = c * B + stripe
    pltpu.sync_copy(t_sh.at[pl.ds(stripe, B // NS)],
                    t_hbm.at[pl.ds(off_out, B // NS)])


def _cumsum(x, axis):
    n = x.shape[axis]
    k = 1
    while k < n:
        shp = list(x.shape)
        shp[axis] = k
        shifted = jnp.concatenate(
            [jnp.zeros(shp, x.dtype), lax.slice_in_dim(x, 0, n - k, axis=axis)],
            axis=axis)
        x = x + shifted
        k *= 2
    return x


def _tc_reduce(n_real, t_ref, out_ref, st_ref):
    g = pl.program_id(0)

    @pl.when(g == 0)
    def _():
        st_ref[0] = 0.0
        st_ref[1] = 0.0

    t0 = t_ref[0, 0]
    t1 = t_ref[1, 0]
    c0 = (t0 + (1 << (CNT_SHIFT - 1))) >> CNT_SHIFT
    c1 = (t1 + (1 << (CNT_SHIFT - 1))) >> CNT_SHIFT
    f0 = t0 - (c0 << CNT_SHIFT)
    f1 = t1 - (c1 << CNT_SHIFT)
    cnt = (c0 + c1).astype(jnp.float32)
    S = (f0 + f1).astype(jnp.float32) * INV_VAL_SCALE
    gi = (g * RB * 128
          + lax.broadcasted_iota(jnp.int32, (RB, 128), 0) * 128
          + lax.broadcasted_iota(jnp.int32, (RB, 128), 1))
    cnt = cnt - jnp.where(gi >= n_real - B, 1.0, 0.0)

    rowsum = jnp.sum(cnt, axis=1, keepdims=True)
    rowpre = _cumsum(rowsum, 0) - rowsum
    colpre = _cumsum(cnt, 1) - cnt
    carry = st_ref[0]
    terms = (2.0 * (carry + rowpre + colpre) + cnt - float(n_real)) * S
    st_ref[0] = carry + jnp.sum(rowsum)
    st_ref[1] = st_ref[1] + jnp.sum(terms)

    @pl.when(g == NB - 1)
    def _():
        out_ref[0, 0] = -st_ref[1] * float(1.0 / (n_real * n_real))


def kernel(y_pred, y_true):
    n = y_pred.shape[0]
    y_true = y_true.reshape(y_pred.shape)
    pad = NP - n
    pad_g = np.arange(n, NP, dtype=np.int64)
    pad_vals = jnp.asarray(
        ((pad_g & (B - 1)).astype(np.float32) + 0.5) * np.float32(1.0 / B))
    pad_zeros = jnp.asarray(np.zeros((pad,), np.float32))

    mesh = plsc.VectorSubcoreMesh(core_axis_name="c", subcore_axis_name="s",
                                  num_cores=NC, num_subcores=NS)
    hist = pl.kernel(
        functools.partial(_sc_hist, n),
        out_type=jax.ShapeDtypeStruct((NC * B,), jnp.int32),
        mesh=mesh,
        scratch_types=[
            pltpu.VMEM((CH,), jnp.float32),
            pltpu.VMEM((CH,), jnp.float32),
            pltpu.VMEM((CH,), jnp.float32),
            pltpu.VMEM((CH,), jnp.float32),
            pltpu.VMEM((ROWS, 128), jnp.int32),
            pltpu.VMEM((CH,), jnp.int32),
            pltpu.VMEM_SHARED((B,), jnp.int32),
            pltpu.SemaphoreType.DMA,
            pltpu.SemaphoreType.DMA,
        ],
    )
    tpk = hist(y_true, y_pred, pad_vals, pad_zeros)

    t4 = tpk.reshape(NC, NB, RB, 128)
    out = pl.pallas_call(
        functools.partial(_tc_reduce, n),
        grid=(NB,),
        in_specs=[pl.BlockSpec((NC, 1, RB, 128), lambda g: (0, g, 0, 0))],
        out_specs=pl.BlockSpec((1, 1), lambda g: (0, 0),
                               memory_space=pltpu.SMEM),
        out_shape=jax.ShapeDtypeStruct((1, 1), jnp.float32),
        scratch_shapes=[pltpu.SMEM((2,), jnp.float32)],
        compiler_params=pltpu.CompilerParams(
            dimension_semantics=("arbitrary",)),
    )(t4)
    return out[0, 0]
